# Initial kernel scaffold; baseline (speedup 1.0000x reference)
import jax
import jax.numpy as jnp
from jax.experimental import pallas as pl


def kernel(aa, E_idx, mask_attend, table):
    raise NotImplementedError("write your pallas kernel here")


# speedup vs baseline: 4.2182x; 4.2182x over previous
import functools

import jax
import jax.numpy as jnp
from jax import lax
from jax.experimental import pallas as pl
from jax.experimental.pallas import tpu as pltpu
from jax.experimental.pallas import tpu_sc as plsc

MAX_AA = 22
K_EMB = 16

_B, _L, _NBR = 4, 2048, 48
_NW = 32
_ROWS_W = (_B * _L) // _NW
_TILES_PER_BATCH = _L // _ROWS_W
_CI = 32
_NCHUNK = _ROWS_W // _CI
_CROWS = _CI * _NBR


def _body(aa_hbm, e_hbm, m_hbm, tblt_hbm, out_hbm,
          aa_v, tblt_v, e_v, m_v, ob0, ob1, sem0, sem1):
    cid = lax.axis_index("c")
    sid = lax.axis_index("s")
    w = sid * 2 + cid
    b = w // _TILES_PER_BATCH
    g0 = w * _ROWS_W
    i0 = (w % _TILES_PER_BATCH) * _ROWS_W

    pltpu.sync_copy(aa_hbm.at[b], aa_v)
    pltpu.sync_copy(tblt_hbm, tblt_v)
    pltpu.sync_copy(e_hbm.at[pl.ds(g0, _ROWS_W)], e_v)
    pltpu.sync_copy(m_hbm.at[pl.ds(g0, _ROWS_W)], m_v)

    iota16 = lax.broadcasted_iota(jnp.int32, (16,), 0)
    dvecs = [jnp.full((16,), d, jnp.int32) for d in range(K_EMB)]

    def chunk_compute(cc, buf):
        def body_i(i, carry):
            irow = cc * _CI + i
            av = plsc.load_gather(aa_v, [jnp.broadcast_to(i0 + irow, (16,))])
            apos = (av + 1) % MAX_AA
            apos22 = apos * MAX_AA
            avalid = apos != 0
            for kb in range(_NBR // 16):
                e16 = e_v[irow, pl.ds(kb * 16, 16)]
                a16 = plsc.load_gather(aa_v, [e16])
                c16 = (a16 + 1) % MAX_AA
                pidx = apos22 + c16
                mk = m_v[irow, pl.ds(kb * 16, 16)]
                mk = jnp.where(avalid & (c16 != 0), mk, 0.0)
                rvec = (i * _NBR + kb * 16) + iota16
                for d in range(K_EMB):
                    col = plsc.load_gather(tblt_v, [dvecs[d], pidx])
                    plsc.store_scatter(buf, [rvec, dvecs[d]], col * mk)
            return carry
        lax.fori_loop(0, _CI, body_i, 0)

    bufs = (ob0, ob1)
    sems = (sem0, sem1)
    descs = [None, None]
    for cc in range(_NCHUNK):
        p = cc % 2
        if descs[p] is not None:
            descs[p].wait()
        chunk_compute(cc, bufs[p])
        dst = out_hbm.at[pl.ds((g0 + cc * _CI) * _NBR, _CROWS)]
        descs[p] = pltpu.async_copy(bufs[p], dst, sems[p])
    descs[0].wait()
    descs[1].wait()


@jax.jit
def kernel(aa, E_idx, mask_attend, table):
    aa32 = aa.astype(jnp.int32)
    e_flat = E_idx.astype(jnp.int32).reshape(_B * _L, _NBR)
    m_flat = mask_attend.reshape(_B * _L, _NBR)
    tblt = table.T

    mesh = plsc.VectorSubcoreMesh(core_axis_name="c", subcore_axis_name="s")
    run = pl.kernel(
        _body,
        out_type=jax.ShapeDtypeStruct((_B * _L * _NBR, K_EMB), jnp.float32),
        mesh=mesh,
        compiler_params=pltpu.CompilerParams(
            needs_layout_passes=False, use_tc_tiling_on_sc=False),
        scratch_types=[
            pltpu.VMEM((_L,), jnp.int32),
            pltpu.VMEM((K_EMB, MAX_AA * MAX_AA), jnp.float32),
            pltpu.VMEM((_ROWS_W, _NBR), jnp.int32),
            pltpu.VMEM((_ROWS_W, _NBR), jnp.float32),
            pltpu.VMEM((_CROWS, K_EMB), jnp.float32),
            pltpu.VMEM((_CROWS, K_EMB), jnp.float32),
            pltpu.SemaphoreType.DMA,
            pltpu.SemaphoreType.DMA,
        ],
    )
    out = run(aa32, e_flat, m_flat, tblt)
    return out.reshape(_B, _L, _NBR, K_EMB)


# speedup vs baseline: 9.0502x; 2.1455x over previous
import jax
import jax.numpy as jnp
from jax import lax
from jax.experimental import pallas as pl
from jax.experimental.pallas import tpu as pltpu
from jax.experimental.pallas import tpu_sc as plsc

MAX_AA = 22
K_EMB = 16

_B, _L, _NBR = 4, 2048, 48
_NW = 32
_IPW = _L // (_NW // _B)
_TPB = _NW // _B
_CI = 64
_NCHUNK = _IPW // _CI


def _body(aa_hbm, e_hbm, m_hbm, tblt_hbm, out_hbm,
          aa_v, tblt_v, ev, mv, ob, sin, sout):
    cid = lax.axis_index("c")
    sid = lax.axis_index("s")
    w = sid * 2 + cid
    b = w // _TPB
    i0 = (w % _TPB) * _IPW

    pltpu.sync_copy(aa_hbm.at[b], aa_v)
    pltpu.sync_copy(tblt_hbm, tblt_v)

    def stage_in(cc, p):
        src_i = pl.ds(i0 + cc * _CI, _CI)
        de = pltpu.async_copy(e_hbm.at[b, :, src_i], ev[p], sin[p])
        dm = pltpu.async_copy(m_hbm.at[b, :, src_i], mv[p], sin[p])
        return de, dm

    dvecs = [jnp.full((16,), d, jnp.int32) for d in range(K_EMB)]

    def chunk_compute(cc, p):
        for ig in range(_CI // 16):
            a16 = aa_v[pl.ds(i0 + cc * _CI + ig * 16, 16)]
            apos = (a16 + 1) % MAX_AA
            apos22 = apos * MAX_AA
            avalid = apos != 0

            def body_k(k, carry):
                e16 = ev[p][k, pl.ds(ig * 16, 16)]
                c16 = (plsc.load_gather(aa_v, [e16]) + 1) % MAX_AA
                pidx = apos22 + c16
                mk = mv[p][k, pl.ds(ig * 16, 16)]
                mk = jnp.where(avalid & (c16 != 0), mk, 0.0)
                for d in range(K_EMB):
                    col = plsc.load_gather(tblt_v, [dvecs[d], pidx])
                    ob[p][k, d, pl.ds(ig * 16, 16)] = col * mk
                return carry
            lax.fori_loop(0, _NBR, body_k, 0)

    in_descs = [stage_in(0, 0), None]
    out_descs = [None, None]
    for cc in range(_NCHUNK):
        p = cc % 2
        if cc + 1 < _NCHUNK:
            in_descs[1 - p] = stage_in(cc + 1, 1 - p)
        for d in in_descs[p]:
            d.wait()
        if out_descs[p] is not None:
            out_descs[p].wait()
        chunk_compute(cc, p)
        dst = out_hbm.at[b, :, :, pl.ds(i0 + cc * _CI, _CI)]
        out_descs[p] = pltpu.async_copy(ob[p], dst, sout[p])
    out_descs[0].wait()
    out_descs[1].wait()


@jax.jit
def kernel(aa, E_idx, mask_attend, table):
    aa32 = aa.astype(jnp.int32)
    et = jnp.transpose(E_idx.astype(jnp.int32), (0, 2, 1))
    mt = jnp.transpose(mask_attend, (0, 2, 1))
    tblt = table.T

    mesh = plsc.VectorSubcoreMesh(core_axis_name="c", subcore_axis_name="s")
    run = pl.kernel(
        _body,
        out_type=jax.ShapeDtypeStruct((_B, _NBR, K_EMB, _L), jnp.float32),
        mesh=mesh,
        compiler_params=pltpu.CompilerParams(
            needs_layout_passes=False, use_tc_tiling_on_sc=False),
        scratch_types=[
            pltpu.VMEM((_L,), jnp.int32),
            pltpu.VMEM((K_EMB, MAX_AA * MAX_AA), jnp.float32),
            [pltpu.VMEM((_NBR, _CI), jnp.int32)] * 2,
            [pltpu.VMEM((_NBR, _CI), jnp.float32)] * 2,
            [pltpu.VMEM((_NBR, K_EMB, _CI), jnp.float32)] * 2,
            [pltpu.SemaphoreType.DMA] * 2,
            [pltpu.SemaphoreType.DMA] * 2,
        ],
    )
    out = run(aa32, et, mt, tblt)
    return jnp.transpose(out, (0, 3, 1, 2))


# speedup vs baseline: 28.4659x; 3.1453x over previous
import jax
import jax.numpy as jnp
from jax import lax
from jax.experimental import pallas as pl
from jax.experimental.pallas import tpu as pltpu
from jax.experimental.pallas import tpu_sc as plsc

MAX_AA = 22
K_EMB = 16

_B, _L, _NBR = 4, 2048, 48
_NW = 32
_TPB = _NW // _B
_KPW = _NBR // _TPB
_TBL_C = 512


def _body(aa_hbm, e_hbm, m_hbm, tblt_hbm, out_hbm,
          aa_v, tblt_v, ev, mv, ob, sin, sout):
    cid = lax.axis_index("c")
    sid = lax.axis_index("s")
    w = sid * 2 + cid
    b = w // _TPB
    k0 = (w % _TPB) * _KPW

    pltpu.sync_copy(aa_hbm.at[b], aa_v)
    pltpu.sync_copy(tblt_hbm, tblt_v)

    iota16 = lax.broadcasted_iota(jnp.int32, (16,), 0)
    zeros = jnp.zeros((16,), jnp.float32)
    z1 = 44 + 22 * iota16
    z2 = 396 + 22 * iota16
    zmask = iota16 < 4
    dvecs = [jnp.full((16,), d, jnp.int32) for d in range(K_EMB)]
    for d in range(K_EMB):
        plsc.store_scatter(tblt_v, [dvecs[d], z1], zeros)
        plsc.store_scatter(tblt_v, [dvecs[d], z2], zeros, mask=zmask)

    def stage_in(kk, p):
        k = k0 + kk
        kb, kr = k // 8, k % 8
        de = pltpu.async_copy(e_hbm.at[b, kb, :, kr, :], ev[p], sin[p])
        dm = pltpu.async_copy(m_hbm.at[b, kb, :, kr, :], mv[p], sin[p])
        return de, dm

    def slab_compute(p, po):
        @plsc.parallel_loop(0, _L // 16, step=1, unroll=4)
        def body_ig(ig):
            ib = ig // 8
            lo = (ig % 8) * 16
            so = pl.ds(lo, 16)
            s = pl.ds(ig * 16, 16)
            u22 = (aa_v[s] + 1) * MAX_AA
            v = plsc.load_gather(aa_v, [ev[p][ib, so]]) + 1
            idx = u22 + v
            mk = mv[p][ib, so]
            for d in range(K_EMB):
                col = plsc.load_gather(tblt_v, [dvecs[d], idx])
                ob[po][d // 8, ib, d % 8, so] = col * mk

    in_descs = [stage_in(0, 0), None]
    out_descs = [None, None, None]
    for kk in range(_KPW):
        p = kk % 2
        po = kk % 3
        if kk + 1 < _KPW:
            in_descs[1 - p] = stage_in(kk + 1, 1 - p)
        for d in in_descs[p]:
            d.wait()
        if out_descs[po] is not None:
            out_descs[po].wait()
        slab_compute(p, po)
        out_descs[po] = pltpu.async_copy(ob[po], out_hbm.at[b, k0 + kk], sout[po])
    for d in out_descs:
        d.wait()


@jax.jit
def kernel(aa, E_idx, mask_attend, table):
    aa32 = aa.astype(jnp.int32)
    e5 = E_idx.astype(jnp.int32).reshape(_B, _L // 128, 128, _NBR // 8, 8)
    e5 = jnp.transpose(e5, (0, 3, 1, 4, 2))
    m5 = mask_attend.reshape(_B, _L // 128, 128, _NBR // 8, 8)
    m5 = jnp.transpose(m5, (0, 3, 1, 4, 2))
    tblt = jnp.pad(table.T, ((0, 0), (0, _TBL_C - MAX_AA * MAX_AA)))

    mesh = plsc.VectorSubcoreMesh(core_axis_name="c", subcore_axis_name="s")
    run = pl.kernel(
        _body,
        out_type=jax.ShapeDtypeStruct(
            (_B, _NBR, K_EMB // 8, _L // 128, 8, 128), jnp.float32),
        mesh=mesh,
        compiler_params=pltpu.CompilerParams(
            needs_layout_passes=False, use_tc_tiling_on_sc=False),
        scratch_types=[
            pltpu.VMEM((_L,), jnp.int32),
            pltpu.VMEM((K_EMB, _TBL_C), jnp.float32),
            [pltpu.VMEM((_L // 128, 128), jnp.int32)] * 2,
            [pltpu.VMEM((_L // 128, 128), jnp.float32)] * 2,
            [pltpu.VMEM((K_EMB // 8, _L // 128, 8, 128), jnp.float32)] * 3,
            [pltpu.SemaphoreType.DMA] * 2,
            [pltpu.SemaphoreType.DMA] * 3,
        ],
    )
    out = run(aa32, e5, m5, tblt)
    out = jnp.transpose(out, (0, 3, 5, 1, 2, 4))
    return out.reshape(_B, _L, _NBR, K_EMB)


# speedup vs baseline: 29.1030x; 1.0224x over previous
import jax
import jax.numpy as jnp
from jax import lax
from jax.experimental import pallas as pl
from jax.experimental.pallas import tpu as pltpu
from jax.experimental.pallas import tpu_sc as plsc

MAX_AA = 22
K_EMB = 16

_B, _L, _NBR = 4, 2048, 48
_NW = 32
_TPB = _NW // _B
_KPW = _NBR // _TPB
_TBL_C = 512


def _body(aa_hbm, e_hbm, m_hbm, tblt_hbm, out_hbm,
          aa_v, tblt_v, ev, mv, ob, sin, sout):
    cid = lax.axis_index("c")
    sid = lax.axis_index("s")
    w = sid * 2 + cid
    b = w // _TPB
    k0 = (w % _TPB) * _KPW

    iota16 = lax.broadcasted_iota(jnp.int32, (16,), 0)
    zeros = jnp.zeros((16,), jnp.float32)
    z1 = 44 + 22 * iota16
    z2 = 396 + 22 * iota16
    zmask = iota16 < 4
    dvecs = [jnp.full((16,), d, jnp.int32) for d in range(K_EMB)]

    def mask_table():
        for d in range(K_EMB):
            plsc.store_scatter(tblt_v, [dvecs[d], z1], zeros)
            plsc.store_scatter(tblt_v, [dvecs[d], z2], zeros, mask=zmask)

    def stage_in(kk, p):
        k = k0 + kk
        kb, kr = k // 8, k % 8
        de = pltpu.async_copy(e_hbm.at[b, kb, :, kr, :], ev[p], sin[p])
        dm = pltpu.async_copy(m_hbm.at[b, kb, :, kr, :], mv[p], sin[p])
        return de, dm

    def slab_compute(p, po):
        @plsc.parallel_loop(0, _L // 16, step=1, unroll=4)
        def body_ig(ig):
            ib = ig // 8
            lo = (ig % 8) * 16
            so = pl.ds(lo, 16)
            s = pl.ds(ig * 16, 16)
            u22 = (aa_v[s] + 1) * MAX_AA
            v = plsc.load_gather(aa_v, [ev[p][ib, so]]) + 1
            idx = u22 + v
            mk = mv[p][ib, so]
            for d in range(K_EMB):
                col = plsc.load_gather(tblt_v, [dvecs[d], idx])
                ob[po][d // 8, ib, d % 8, so] = col * mk

    in_descs = [stage_in(0, 0), None]
    da = pltpu.async_copy(aa_hbm.at[b], aa_v, sout[2])
    dt = pltpu.async_copy(tblt_hbm, tblt_v, sout[2])
    da.wait()
    dt.wait()
    mask_table()
    out_descs = [None, None, None]
    for kk in range(_KPW):
        p = kk % 2
        po = kk % 3
        if kk + 1 < _KPW:
            in_descs[1 - p] = stage_in(kk + 1, 1 - p)
        for d in in_descs[p]:
            d.wait()
        if out_descs[po] is not None:
            out_descs[po].wait()
        slab_compute(p, po)
        out_descs[po] = pltpu.async_copy(ob[po], out_hbm.at[b, k0 + kk], sout[po])
    for d in out_descs:
        d.wait()


@jax.jit
def kernel(aa, E_idx, mask_attend, table):
    aa32 = aa.astype(jnp.int32)
    e5 = E_idx.astype(jnp.int32).reshape(_B, _L // 128, 128, _NBR // 8, 8)
    e5 = jnp.transpose(e5, (0, 3, 1, 4, 2))
    m5 = mask_attend.reshape(_B, _L // 128, 128, _NBR // 8, 8)
    m5 = jnp.transpose(m5, (0, 3, 1, 4, 2))
    tblt = jnp.pad(table.T, ((0, 0), (0, _TBL_C - MAX_AA * MAX_AA)))

    mesh = plsc.VectorSubcoreMesh(core_axis_name="c", subcore_axis_name="s")
    run = pl.kernel(
        _body,
        out_type=jax.ShapeDtypeStruct(
            (_B, _NBR, K_EMB // 8, _L // 128, 8, 128), jnp.float32),
        mesh=mesh,
        compiler_params=pltpu.CompilerParams(
            needs_layout_passes=False, use_tc_tiling_on_sc=False),
        scratch_types=[
            pltpu.VMEM((_L,), jnp.int32),
            pltpu.VMEM((K_EMB, _TBL_C), jnp.float32),
            [pltpu.VMEM((_L // 128, 128), jnp.int32)] * 2,
            [pltpu.VMEM((_L // 128, 128), jnp.float32)] * 2,
            [pltpu.VMEM((K_EMB // 8, _L // 128, 8, 128), jnp.float32)] * 3,
            [pltpu.SemaphoreType.DMA] * 2,
            [pltpu.SemaphoreType.DMA] * 3,
        ],
    )
    out = run(aa32, e5, m5, tblt)
    out = jnp.transpose(out, (0, 3, 5, 1, 2, 4))
    return out.reshape(_B, _L, _NBR, K_EMB)
